# baseline (speedup 1.0000x reference)
import functools

import jax
import jax.numpy as jnp
from jax import lax
from jax.experimental import pallas as pl
from jax.experimental.pallas import tpu as pltpu
from jax.experimental.pallas import tpu_sc as plsc

_U = 10000
_I = 5000
_G = 2000
_D = 64
_LG_ITEM = 3000
_B = 16384
_F32 = jnp.float32


def _overlap_body(*args):
    a_parts, (g_ref, out_ref, c1_ref) = args[:_OV_SPLIT], args[_OV_SPLIT:]
    q = _G // _OV_SPLIT
    parts = a_parts
    g = g_ref[...]
    for k, a_ref in enumerate(parts):
        c1_ref[pl.ds(k * q, q), :] = jnp.dot(a_ref[...], g,
                                             preferred_element_type=_F32)
    c1 = c1_ref[...]
    for k, a_ref in enumerate(parts):
        sl = pl.ds(k * q, q)
        out_ref[sl, :] = (g_ref[sl, :] + c1_ref[sl, :]
                          + jnp.dot(a_ref[...], c1,
                                    preferred_element_type=_F32))


_OV_SPLIT = 5


def _overlap_conv(overlap_graph, group_table):
    q = _G // _OV_SPLIT
    return pl.pallas_call(
        _overlap_body,
        grid=(1,),
        in_specs=[pl.BlockSpec((q, _G), lambda i, k=k: (k, 0))
                  for k in range(_OV_SPLIT)]
        + [pl.BlockSpec((_G, _D), lambda i: (0, 0))],
        out_specs=pl.BlockSpec((_G, _D), lambda i: (0, 0)),
        out_shape=jax.ShapeDtypeStruct((_G, _D), _F32),
        scratch_shapes=[pltpu.VMEM((_G, _D), _F32)],
    )(*([overlap_graph] * _OV_SPLIT + [group_table]))


def _msg_body(uh_ref, ih_ref, u_ref, it_ref, ge_ref, w_ref, b_ref, out_ref):
    um = jnp.dot(uh_ref[...], u_ref[...], preferred_element_type=_F32)
    im = jnp.dot(ih_ref[...], it_ref[...], preferred_element_type=_F32)
    ige = im * ge_ref[...]
    w = w_ref[0]
    msg = (jnp.dot(um, w[0:_D], preferred_element_type=_F32)
           + jnp.dot(im, w[_D:2 * _D], preferred_element_type=_F32)
           + jnp.dot(ige, w[2 * _D:3 * _D], preferred_element_type=_F32)
           + b_ref[0])
    out_ref[...] = msg


def _msg_layer(user_hyper, item_hyper, u_arr, it_arr, it_block_idx, layer,
               group_emb, W_agg, b_agg):
    bm = 200
    grid = (_G // bm,)
    return pl.pallas_call(
        _msg_body,
        grid=grid,
        in_specs=[
            pl.BlockSpec((bm, _U), lambda i: (i, 0)),
            pl.BlockSpec((bm, _I), lambda i: (i, 0)),
            pl.BlockSpec((_U, _D), lambda i: (0, 0)),
            pl.BlockSpec((_I, _D), lambda i: (it_block_idx, 0)),
            pl.BlockSpec((bm, _D), lambda i: (i, 0)),
            pl.BlockSpec((1, 3 * _D, _D), lambda i: (layer, 0, 0)),
            pl.BlockSpec((1, 1, _D), lambda i: (layer, 0, 0)),
        ],
        out_specs=pl.BlockSpec((bm, _D), lambda i: (i, 0)),
        out_shape=jax.ShapeDtypeStruct((_G, _D), _F32),
    )(user_hyper, item_hyper, u_arr, it_arr, group_emb, W_agg, b_agg)


_FH_BM = 1000
_FH_SPLIT = 5
_FH_Q = _FH_BM // _FH_SPLIT


def _fh0_body(*args):
    f_refs, (msg_ref, out_ref) = args[:_FH_SPLIT], args[_FH_SPLIT:]
    msg = msg_ref[...]
    for k, f_ref in enumerate(f_refs):
        out_ref[pl.ds(k * _FH_Q, _FH_Q), :] = jnp.dot(
            f_ref[...], msg, preferred_element_type=_F32)


def _fh_layer0(full_hyper, msg):
    grid = ((_U + _I) // _FH_BM,)
    return pl.pallas_call(
        _fh0_body,
        grid=grid,
        in_specs=[pl.BlockSpec((_FH_Q, _G),
                               lambda i, k=k: (_FH_SPLIT * i + k, 0))
                  for k in range(_FH_SPLIT)]
        + [pl.BlockSpec((_G, _D), lambda i: (0, 0))],
        out_specs=pl.BlockSpec((_FH_BM, _D), lambda i: (i, 0)),
        out_shape=jax.ShapeDtypeStruct((_U + _I, _D), _F32),
    )(*([full_hyper] * _FH_SPLIT + [msg]))


def _fh1_body(*args):
    f_refs = args[:_FH_SPLIT]
    (msg_ref, it_ref, n0_ref, ge_ref, m0_ref, lgem_ref, wov_ref, bov_ref,
     why_ref, bhy_ref, wlg_ref, blg_ref, iemb_ref, gui_ref) = args[_FH_SPLIT:]
    msg = msg_ref[...]
    for k, f_ref in enumerate(f_refs):
        sl = pl.ds(k * _FH_Q, _FH_Q)
        iemb_ref[sl, :] = (it_ref[sl, :] + n0_ref[sl, :]
                           + jnp.dot(f_ref[...], msg,
                                     preferred_element_type=_F32))

    @pl.when(pl.program_id(0) == 0)
    def _():
        ge = ge_ref[...]
        he = ge + m0_ref[...] + msg
        lg = lgem_ref[...]
        co = jax.nn.sigmoid(jnp.dot(ge, wov_ref[...],
                                    preferred_element_type=_F32) + bov_ref[...])
        ch = jax.nn.sigmoid(jnp.dot(he, why_ref[...],
                                    preferred_element_type=_F32) + bhy_ref[...])
        cl = jax.nn.sigmoid(jnp.dot(lg, wlg_ref[...],
                                    preferred_element_type=_F32) + blg_ref[...])
        gui_ref[...] = co * ge + ch * he + cl * lg


def _fh_layer1_items(full_hyper, msg1, item_table, norm0, group_emb, msg0,
                     lg_emb, wov, bov, why, bhy, wlg, blg):
    off = _U // _FH_Q
    grid = (_I // _FH_BM,)
    return pl.pallas_call(
        _fh1_body,
        grid=grid,
        in_specs=[pl.BlockSpec((_FH_Q, _G),
                               lambda i, k=k: (off + _FH_SPLIT * i + k, 0))
                  for k in range(_FH_SPLIT)]
        + [
            pl.BlockSpec((_G, _D), lambda i: (0, 0)),
            pl.BlockSpec((_FH_BM, _D), lambda i: (i, 0)),
            pl.BlockSpec((_FH_BM, _D), lambda i: (i + _U // _FH_BM, 0)),
            pl.BlockSpec((_G, _D), lambda i: (0, 0)),
            pl.BlockSpec((_G, _D), lambda i: (0, 0)),
            pl.BlockSpec((_G, _D), lambda i: (0, 0)),
            pl.BlockSpec((_D, 1), lambda i: (0, 0)),
            pl.BlockSpec((1,), lambda i: (0,)),
            pl.BlockSpec((_D, 1), lambda i: (0, 0)),
            pl.BlockSpec((1,), lambda i: (0,)),
            pl.BlockSpec((_D, 1), lambda i: (0, 0)),
            pl.BlockSpec((1,), lambda i: (0,)),
        ],
        out_specs=[
            pl.BlockSpec((_FH_BM, _D), lambda i: (i, 0)),
            pl.BlockSpec((_G, _D), lambda i: (0, 0)),
        ],
        out_shape=[
            jax.ShapeDtypeStruct((_I, _D), _F32),
            jax.ShapeDtypeStruct((_G, _D), _F32),
        ],
    )(*([full_hyper] * _FH_SPLIT
        + [msg1, item_table, norm0, group_emb, msg0, lg_emb,
           wov, bov, why, bhy, wlg, blg]))


def _lg1_body(lg_ref, g_ref, it_ref, out_ref, e0_ref):
    @pl.when(pl.program_id(0) == 0)
    def _():
        e0_ref[0:_G, :] = g_ref[...]
        e0_ref[_G:, :] = it_ref[...]

    out_ref[...] = jnp.dot(lg_ref[...], e0_ref[...],
                           preferred_element_type=_F32)


def _lgcn_layer1(lgcn_graph, group_table, item_table):
    n = _G + _LG_ITEM
    bm = 200
    grid = (n // bm,)
    return pl.pallas_call(
        _lg1_body,
        grid=grid,
        in_specs=[
            pl.BlockSpec((bm, n), lambda i: (i, 0)),
            pl.BlockSpec((_G, _D), lambda i: (0, 0)),
            pl.BlockSpec((_LG_ITEM, _D), lambda i: (0, 0)),
        ],
        out_specs=pl.BlockSpec((bm, _D), lambda i: (i, 0)),
        out_shape=jax.ShapeDtypeStruct((n, _D), _F32),
        scratch_shapes=[pltpu.VMEM((n, _D), _F32)],
    )(lgcn_graph, group_table, item_table)


def _lg2_body(lg_ref, c1_ref, g_blk_ref, c1_blk_ref, out_ref):
    c2 = jnp.dot(lg_ref[...], c1_ref[...], preferred_element_type=_F32)
    out_ref[...] = (g_blk_ref[...] + c1_blk_ref[...] + c2) * (1.0 / 3.0)


def _lgcn_layer2_groups(lgcn_graph, cur1, group_table):
    n = _G + _LG_ITEM
    bm = 200
    grid = (_G // bm,)
    return pl.pallas_call(
        _lg2_body,
        grid=grid,
        in_specs=[
            pl.BlockSpec((bm, n), lambda i: (i, 0)),
            pl.BlockSpec((n, _D), lambda i: (0, 0)),
            pl.BlockSpec((bm, _D), lambda i: (i, 0)),
            pl.BlockSpec((bm, _D), lambda i: (i, 0)),
        ],
        out_specs=pl.BlockSpec((bm, _D), lambda i: (i, 0)),
        out_shape=jax.ShapeDtypeStruct((_G, _D), _F32),
    )(lgcn_graph, cur1, group_table, cur1)


def _dot_body(g_ref, i_ref, out_ref):
    out_ref[...] = jnp.sum(g_ref[...] * i_ref[...], axis=1)


def _pair_dot(g_sel, i_sel):
    bm = 4096
    grid = (_B // bm,)
    return pl.pallas_call(
        _dot_body,
        grid=grid,
        in_specs=[
            pl.BlockSpec((bm, _D), lambda i: (i, 0)),
            pl.BlockSpec((bm, _D), lambda i: (i, 0)),
        ],
        out_specs=pl.BlockSpec((bm,), lambda i: (i,)),
        out_shape=jax.ShapeDtypeStruct((_B,), _F32),
    )(g_sel, i_sel)


_NC = 2
_NS = 16
_NW = _NC * _NS
_BPW = _B // _NW


def _sc_gather_pair(g_tab, i_tab, g_idx, i_idx):
    mesh = plsc.VectorSubcoreMesh(core_axis_name="c", subcore_axis_name="s")

    @functools.partial(
        pl.kernel,
        mesh=mesh,
        out_type=[
            jax.ShapeDtypeStruct((_B, _D), _F32),
            jax.ShapeDtypeStruct((_B, _D), _F32),
        ],
        scratch_types=[
            pltpu.VMEM((_BPW,), jnp.int32),
            pltpu.VMEM((_BPW, _D), _F32),
            pltpu.SemaphoreType.DMA,
        ],
        compiler_params=pltpu.CompilerParams(use_tc_tiling_on_sc=False),
    )
    def k(g_tab_hbm, i_tab_hbm, gidx_hbm, iidx_hbm, gout_hbm, iout_hbm,
          idx_v, rows_v, sem):
        wid = lax.axis_index("s") * _NC + lax.axis_index("c")
        base = wid * _BPW
        pltpu.sync_copy(gidx_hbm.at[pl.ds(base, _BPW)], idx_v)
        pltpu.async_copy(g_tab_hbm.at[idx_v], rows_v, sem).wait()
        pltpu.sync_copy(rows_v, gout_hbm.at[pl.ds(base, _BPW)])
        pltpu.sync_copy(iidx_hbm.at[pl.ds(base, _BPW)], idx_v)
        pltpu.async_copy(i_tab_hbm.at[idx_v], rows_v, sem).wait()
        pltpu.sync_copy(rows_v, iout_hbm.at[pl.ds(base, _BPW)])

    return k(g_tab, i_tab, g_idx, i_idx)


def kernel(user_table, item_table, group_table, user_hyper, item_hyper,
           full_hyper, overlap_graph, lgcn_graph, W_agg, b_agg,
           W_ov, b_ov, W_hy, b_hy, W_lg, b_lg,
           group_inputs, item_inputs):
    group_emb = _overlap_conv(overlap_graph, group_table)

    cur1 = _lgcn_layer1(lgcn_graph, group_table, item_table)
    lg_emb = _lgcn_layer2_groups(lgcn_graph, cur1, group_table)

    b_agg3 = b_agg.reshape(2, 1, _D)
    msg0 = _msg_layer(user_hyper, item_hyper, user_table, item_table, 0, 0,
                      group_emb, W_agg, b_agg3)
    norm0 = _fh_layer0(full_hyper, msg0)

    msg1 = _msg_layer(user_hyper, item_hyper, norm0, norm0, 2, 1,
                      group_emb, W_agg, b_agg3)
    i_emb_full, group_ui_emb = _fh_layer1_items(
        full_hyper, msg1, item_table, norm0, group_emb, msg0, lg_emb,
        W_ov, b_ov, W_hy, b_hy, W_lg, b_lg)

    g_sel, i_sel = _sc_gather_pair(group_ui_emb, i_emb_full,
                                   group_inputs, item_inputs)
    return _pair_dot(g_sel, i_sel)
